# baseline (speedup 1.0000x reference)
import functools

import jax
import jax.numpy as jnp
from jax import lax
from jax.experimental import pallas as pl
from jax.experimental.pallas import tpu as pltpu
from jax.experimental.pallas import tpu_sc as plsc

B = 16384
EMB = 32
HID = 64
PACK = 4
VOCAB = 1000000
NC = 2
NS = 16
NW = NC * NS
CHUNK = 128
NCHUNK = B // CHUNK
CPW = NCHUNK // NW
L = 16
SZW = CPW * CHUNK * EMB

_mesh = plsc.VectorSubcoreMesh(core_axis_name="c", subcore_axis_name="s")


def _extract(idx_ref, j, wide_ref, out_ref):
    def body(g, _):
        offv = (idx_ref[j, pl.ds(g * L, L)] & (PACK - 1)) << 5
        gbase = g * (L * EMB)
        for k in range(L):
            off = offv[k]
            dst = gbase + (j * CHUNK + k) * EMB
            out_ref[pl.ds(dst, L)] = wide_ref[g * L + k, pl.ds(off, L)]
            out_ref[pl.ds(dst + L, L)] = wide_ref[g * L + k, pl.ds(off + L, L)]
        return 0
    lax.fori_loop(0, CHUNK // L, body, 0)


@functools.partial(
    pl.kernel,
    out_type=jax.ShapeDtypeStruct((2, NW, SZW), jnp.float32),
    mesh=_mesh,
    compiler_params=pltpu.CompilerParams(use_tc_tiling_on_sc=False),
    scratch_types=[
        pltpu.VMEM((CPW, CHUNK), jnp.int32),
        pltpu.VMEM((CPW, CHUNK), jnp.int32),
        pltpu.VMEM((CPW, CHUNK), jnp.int32),
        pltpu.VMEM((CPW, CHUNK), jnp.int32),
        pltpu.VMEM((2, CHUNK, EMB * PACK), jnp.float32),
        pltpu.VMEM((SZW,), jnp.float32),
        pltpu.VMEM((SZW,), jnp.float32),
        pltpu.SemaphoreType.DMA,
        pltpu.SemaphoreType.DMA,
    ],
)
def _sc_gather(idx_hbm, lon_hbm, lat_hbm, out_hbm,
               idx_lon, idx_lat, row_lon, row_lat, wide, ext_lon, ext_lat,
               sem0, sem1):
    wid = lax.axis_index("s") * NC + lax.axis_index("c")
    base = wid * CPW
    pltpu.sync_copy(idx_hbm.at[0, pl.ds(base, CPW)], idx_lon)
    pltpu.sync_copy(idx_hbm.at[1, pl.ds(base, CPW)], idx_lat)
    for j in range(CPW):
        for g in range(CHUNK // L):
            sl = pl.ds(g * L, L)
            row_lon[j, sl] = lax.shift_right_logical(idx_lon[j, sl], 2)
            row_lat[j, sl] = lax.shift_right_logical(idx_lat[j, sl], 2)

    units = [(t, j) for j in range(CPW) for t in range(2)]
    sems = (sem0, sem1)

    def fire(u, buf):
        t, j = units[u]
        src = lon_hbm if t == 0 else lat_hbm
        rows = row_lon if t == 0 else row_lat
        return pltpu.async_copy(src.at[rows.at[j]], wide.at[buf], sems[buf])

    pending = [fire(0, 0), fire(1, 1)]
    for u in range(len(units)):
        buf = u % 2
        pending[buf].wait()
        t, j = units[u]
        idx_ref = idx_lon if t == 0 else idx_lat
        ext_ref = ext_lon if t == 0 else ext_lat
        _extract(idx_ref, j, wide.at[buf], ext_ref)
        if u + 2 < len(units):
            pending[buf] = fire(u + 2, buf)
    pltpu.sync_copy(ext_lon, out_hbm.at[0, wid])
    pltpu.sync_copy(ext_lat, out_hbm.at[1, wid])


BM = 2048


def _mm_body(x_ref, wt_ref, b_ref, o_ref):
    acc = jnp.dot(x_ref[0], wt_ref[:EMB, :], preferred_element_type=jnp.float32)
    acc = acc + jnp.dot(x_ref[1], wt_ref[EMB:, :], preferred_element_type=jnp.float32)
    o_ref[...] = acc + b_ref[...]


def _tc_project(emb, wt, b2):
    return pl.pallas_call(
        _mm_body,
        grid=(B // BM,),
        in_specs=[
            pl.BlockSpec((2, BM, EMB), lambda i: (0, i, 0)),
            pl.BlockSpec((2 * EMB, HID), lambda i: (0, 0)),
            pl.BlockSpec((1, HID), lambda i: (0, 0)),
        ],
        out_specs=pl.BlockSpec((BM, HID), lambda i: (i, 0)),
        out_shape=jax.ShapeDtypeStruct((B, HID), jnp.float32),
    )(emb, wt, b2)


def kernel(batch_seq_cat, lon_table, lat_table, W, b):
    idx_t = batch_seq_cat.T.reshape(2, NCHUNK, CHUNK)
    lon4 = lon_table.reshape(VOCAB // PACK, EMB * PACK)
    lat4 = lat_table.reshape(VOCAB // PACK, EMB * PACK)
    emb = _sc_gather(idx_t, lon4, lat4).reshape(2, B, EMB)
    return _tc_project(emb, W.T, b.reshape(1, HID))
